# baseline (speedup 1.0000x reference)
import functools

import jax
import jax.numpy as jnp
from jax import lax
from jax.experimental import pallas as pl
from jax.experimental.pallas import tpu as pltpu
from jax.experimental.pallas import tpu_sc as plsc

LANES = 16
NC = 2
NS = 16
K = 1024
SLAB = 8


def _build_rank_kernel(B, S, H, W):
    QR = H // 4
    GPQ = QR * W // LANES
    mesh = plsc.VectorSubcoreMesh(core_axis_name="c", subcore_axis_name="s")

    @functools.partial(
        pl.kernel,
        out_type=(jax.ShapeDtypeStruct((S, 1, K), jnp.int32),
                  jax.ShapeDtypeStruct((NC, NS, 1, K), jnp.int32),
                  jax.ShapeDtypeStruct((NC, 2, 2, 4, 1, K), jnp.float32),
                  jax.ShapeDtypeStruct((B, S, H, W), jnp.int32)),
        mesh=mesh,
        scratch_types=[
            pltpu.VMEM((QR, W), jnp.int32),
            pltpu.VMEM((1, K), jnp.int32),
            pltpu.VMEM((8, 1, K), jnp.int32),
            pltpu.VMEM((1, K), jnp.int32),
            pltpu.VMEM((1, K), jnp.float32),
        ],
        compiler_params=pltpu.CompilerParams(needs_layout_passes=False),
    )
    def rank_kernel(seg_hbm, rank_hbm, pres_hbm, cntp_hbm, segs_hbm,
                    seg_v, pres_v, red_v, rank_v, cnt_v):
        ci = lax.axis_index("c")
        si = lax.axis_index("s")
        s_local = si // 8
        b = (si % 8) // 4
        q = si % 4
        s = ci * 2 + s_local
        zero = jnp.zeros((LANES,), jnp.int32)
        ones = jnp.ones((LANES,), jnp.int32)
        onesf = jnp.ones((LANES,), jnp.float32)
        zf = jnp.zeros((LANES,), jnp.float32)
        zconst = jnp.zeros((LANES,), jnp.int32)
        WG = W // LANES
        r0 = pl.multiple_of(q * QR, SLAB)

        pltpu.sync_copy(seg_hbm.at[b, s, pl.ds(r0, QR), pl.ds(0, W)], seg_v)

        def zbody(j, carry):
            jo = pl.multiple_of(j * LANES, LANES)
            pres_v[0, pl.ds(jo, LANES)] = zero
            cnt_v[0, pl.ds(jo, LANES)] = zf
            return carry
        lax.fori_loop(0, K // LANES, zbody, 0)

        @plsc.parallel_loop(0, GPQ, 1, unroll=2)
        def pbody(g):
            r = g // WG
            jo = pl.multiple_of((g % WG) * LANES, LANES)
            v = seg_v[r, pl.ds(jo, LANES)]
            plsc.store_scatter(pres_v, [zconst, v], ones)

        pltpu.sync_copy(pres_v, pres_hbm.at[ci, si])
        plsc.subcore_barrier()

        @pl.when(si % 8 == 0)
        def _():
            pltpu.sync_copy(pres_hbm.at[ci, pl.ds(s_local * 8, 8)], red_v)

            def rbody(j, off):
                jo = pl.multiple_of(j * LANES, LANES)
                tot = red_v[0, 0, pl.ds(jo, LANES)]
                for r in range(1, 8):
                    tot = tot + red_v[r, 0, pl.ds(jo, LANES)]
                p = (tot > 0).astype(jnp.int32)
                incl = plsc.cumsum(p)
                rank_v[0, pl.ds(jo, LANES)] = (incl - p) + off
                return off + jnp.sum(p)
            lax.fori_loop(0, K // LANES, rbody, jnp.int32(0))
            pltpu.sync_copy(rank_v, rank_hbm.at[s])

        plsc.subcore_barrier()
        pltpu.sync_copy(rank_hbm.at[s], rank_v)

        @plsc.parallel_loop(0, GPQ, 1, unroll=2)
        def mbody(g):
            r = g // WG
            jo = pl.multiple_of((g % WG) * LANES, LANES)
            v = seg_v[r, pl.ds(jo, LANES)]
            inv = plsc.load_gather(rank_v, [zconst, v])
            seg_v[r, pl.ds(jo, LANES)] = inv
            plsc.addupdate_scatter(cnt_v, [zconst, inv], onesf)

        pltpu.sync_copy(seg_v,
                        segs_hbm.at[b, s, pl.ds(r0, QR), pl.ds(0, W)])
        pltpu.sync_copy(cnt_v, cntp_hbm.at[ci, s_local, b, q])

    return rank_kernel


def _build_scatter_kernel(B, C, S, H, W):
    CPT = C // NS
    NSLAB = H // SLAB
    GPS = SLAB * W // LANES
    mesh = plsc.VectorSubcoreMesh(core_axis_name="c", subcore_axis_name="s")

    @functools.partial(
        pl.kernel,
        out_type=jax.ShapeDtypeStruct((B, S, NS, 8, 1, K), jnp.float32),
        mesh=mesh,
        scratch_types=[
            pltpu.VMEM((S * 8 * K,), jnp.float32),
            pltpu.VMEM((2, S, SLAB, W), jnp.int32),
            pltpu.VMEM((2, CPT, SLAB, W), jnp.float32),
            pltpu.SemaphoreType.DMA,
            pltpu.SemaphoreType.DMA,
        ],
        compiler_params=pltpu.CompilerParams(needs_layout_passes=False),
    )
    def scatter_kernel(x_hbm, seg_hbm, sums_hbm,
                       acc_v, seg_v, x_v,
                       sem_seg, sem_x):
        b = lax.axis_index("c")
        cb = lax.axis_index("s")
        zf = jnp.zeros((LANES,), jnp.float32)
        WG = W // LANES

        def zbody(j, carry):
            jo = pl.multiple_of(j * LANES, LANES)
            for blk in range(S * 8):
                acc_v[pl.ds(blk * K + jo, LANES)] = zf
            return carry
        lax.fori_loop(0, K // LANES, zbody, 0)

        def seg_copy(slab, buf):
            r0 = pl.multiple_of(slab * SLAB, SLAB)
            return pltpu.make_async_copy(
                seg_hbm.at[b, pl.ds(0, S), pl.ds(r0, SLAB), pl.ds(0, W)],
                seg_v.at[buf], sem_seg)

        def x_copy(slab, buf):
            r0 = pl.multiple_of(slab * SLAB, SLAB)
            return pltpu.make_async_copy(
                x_hbm.at[b, pl.ds(cb * CPT, CPT), pl.ds(r0, SLAB),
                         pl.ds(0, W)],
                x_v.at[buf], sem_x)

        seg_copy(0, 0).start()
        x_copy(0, 0).start()

        def slab_body(slab, carry):
            parity = lax.rem(slab, 2)

            @pl.when(slab + 1 < NSLAB)
            def _():
                seg_copy(slab + 1, 1 - parity).start()
                x_copy(slab + 1, 1 - parity).start()

            seg_copy(slab, parity).wait()
            x_copy(slab, parity).wait()
            seg_b = seg_v.at[parity]
            x_b = x_v.at[parity]

            @plsc.parallel_loop(0, GPS, 1, unroll=6)
            def gbody(g):
                r = g // WG
                jo = pl.multiple_of((g % WG) * LANES, LANES)
                idxs = [seg_b[s, r, pl.ds(jo, LANES)] for s in range(S)]
                for c in range(CPT):
                    xv = x_b[c, r, pl.ds(jo, LANES)]
                    for s in range(S):
                        plsc.addupdate_scatter(
                            acc_v.at[pl.ds((s * 8 + c) * K, K)],
                            [idxs[s]], xv)
            return carry
        lax.fori_loop(0, NSLAB, slab_body, 0)

        for s in range(S):
            for c in range(8):
                pltpu.sync_copy(acc_v.at[pl.ds((s * 8 + c) * K, K)],
                                sums_hbm.at[b, s, cb, c, 0])

    return scatter_kernel


def _tokens_tc(sums, cnts, cls_token, cls_pos_embed, wt, wt_pad, bias2d):
    B, S, C2, _ = sums.shape
    C = wt.shape[0]
    E = wt.shape[1]

    def body(sums_ref, cnts_ref, cls_ref, clsp_ref, wt_ref, wtp_ref, b_ref,
             out_ref):
        sc = jnp.reshape(sums_ref[...], (C2, K))
        cnt4 = jnp.reshape(cnts_ref[...], (4, K))
        cnt = jnp.sum(cnt4, axis=0, keepdims=True)
        recip = 1.0 / jnp.maximum(cnt, 1.0)
        mean = sc * recip
        body_out = lax.dot_general(mean, wtp_ref[...], (((0,), (0,)), ((), ())),
                                   preferred_element_type=jnp.float32)
        cls_row = jnp.dot(cls_ref[...] + clsp_ref[...], wt_ref[...],
                          preferred_element_type=jnp.float32)
        bias = b_ref[...]
        full = jnp.concatenate([cls_row + bias, body_out + bias], axis=0)
        out_ref[...] = jnp.reshape(full, (1, 1, K + 1, E))

    return pl.pallas_call(
        body,
        grid=(S, B),
        in_specs=[
            pl.BlockSpec((1, 1, C2, K), lambda s, b: (b, s, 0, 0)),
            pl.BlockSpec((1, 1, 1, 4, 1, K),
                         lambda s, b: (s // 2, s % 2, b, 0, 0, 0)),
            pl.BlockSpec((1, C), lambda s, b: (0, 0)),
            pl.BlockSpec((1, C), lambda s, b: (0, 0)),
            pl.BlockSpec((C, E), lambda s, b: (0, 0)),
            pl.BlockSpec((C2, E), lambda s, b: (0, 0)),
            pl.BlockSpec((1, E), lambda s, b: (0, 0)),
        ],
        out_specs=pl.BlockSpec((1, 1, K + 1, E), lambda s, b: (s, b, 0, 0)),
        out_shape=jax.ShapeDtypeStruct((S, B, K + 1, E), jnp.float32),
    )(sums, cnts, cls_token, cls_pos_embed, wt, wt_pad, bias2d)


def kernel(x, segments, cls_token, cls_pos_embed, conv_w, conv_b):
    B, C, H, W = x.shape
    S = segments.shape[1]
    CPT = C // NS

    rank, _, cnts, segs = _build_rank_kernel(B, S, H, W)(segments)
    sums6 = _build_scatter_kernel(B, C, S, H, W)(x, segs)
    sums = sums6.reshape(B, S, NS * 8, K)

    wt = conv_w.T
    E = conv_w.shape[0]
    rows = jnp.arange(C, dtype=jnp.int32)
    pad_rows = (rows // CPT) * 8 + (rows % CPT)
    wt_pad = jnp.zeros((NS * 8, E), jnp.float32).at[pad_rows].set(wt)

    toks = _tokens_tc(sums, cnts, cls_token, cls_pos_embed,
                      wt, wt_pad, conv_b.reshape(1, -1))
    tokens = tuple(toks[i] for i in range(S))
    return (tokens, segs)


# speedup vs baseline: 1.0041x; 1.0041x over previous
import functools

import jax
import jax.numpy as jnp
from jax import lax
from jax.experimental import pallas as pl
from jax.experimental.pallas import tpu as pltpu
from jax.experimental.pallas import tpu_sc as plsc

LANES = 16
NC = 2
NS = 16
K = 1024
SLAB = 8


def _build_rank_kernel(B, S, H, W):
    QR = H // 4
    GPQ = QR * W // LANES
    mesh = plsc.VectorSubcoreMesh(core_axis_name="c", subcore_axis_name="s")

    @functools.partial(
        pl.kernel,
        out_type=(jax.ShapeDtypeStruct((S, 1, K), jnp.int32),
                  jax.ShapeDtypeStruct((NC, NS, 1, K), jnp.int32),
                  jax.ShapeDtypeStruct((NC, 2, 2, 4, 1, K), jnp.float32),
                  jax.ShapeDtypeStruct((B, S, H, W), jnp.int32)),
        mesh=mesh,
        scratch_types=[
            pltpu.VMEM((QR, W), jnp.int32),
            pltpu.VMEM((1, K), jnp.int32),
            pltpu.VMEM((8, 1, K), jnp.int32),
            pltpu.VMEM((1, K), jnp.int32),
            pltpu.VMEM((1, K), jnp.float32),
        ],
        compiler_params=pltpu.CompilerParams(needs_layout_passes=False),
    )
    def rank_kernel(seg_hbm, rank_hbm, pres_hbm, cntp_hbm, segs_hbm,
                    seg_v, pres_v, red_v, rank_v, cnt_v):
        ci = lax.axis_index("c")
        si = lax.axis_index("s")
        s_local = si // 8
        b = (si % 8) // 4
        q = si % 4
        s = ci * 2 + s_local
        zero = jnp.zeros((LANES,), jnp.int32)
        ones = jnp.ones((LANES,), jnp.int32)
        onesf = jnp.ones((LANES,), jnp.float32)
        zf = jnp.zeros((LANES,), jnp.float32)
        zconst = jnp.zeros((LANES,), jnp.int32)
        WG = W // LANES
        r0 = pl.multiple_of(q * QR, SLAB)

        pltpu.sync_copy(seg_hbm.at[b, s, pl.ds(r0, QR), pl.ds(0, W)], seg_v)

        def zbody(j, carry):
            jo = pl.multiple_of(j * LANES, LANES)
            pres_v[0, pl.ds(jo, LANES)] = zero
            cnt_v[0, pl.ds(jo, LANES)] = zf
            return carry
        lax.fori_loop(0, K // LANES, zbody, 0)

        @plsc.parallel_loop(0, GPQ, 1, unroll=4)
        def pbody(g):
            r = g // WG
            jo = pl.multiple_of((g % WG) * LANES, LANES)
            v = seg_v[r, pl.ds(jo, LANES)]
            plsc.store_scatter(pres_v, [zconst, v], ones)

        pltpu.sync_copy(pres_v, pres_hbm.at[ci, si])
        plsc.subcore_barrier()

        @pl.when(si % 8 == 0)
        def _():
            pltpu.sync_copy(pres_hbm.at[ci, pl.ds(s_local * 8, 8)], red_v)

            def rbody(j, off):
                jo = pl.multiple_of(j * LANES, LANES)
                tot = red_v[0, 0, pl.ds(jo, LANES)]
                for r in range(1, 8):
                    tot = tot + red_v[r, 0, pl.ds(jo, LANES)]
                p = (tot > 0).astype(jnp.int32)
                incl = plsc.cumsum(p)
                rank_v[0, pl.ds(jo, LANES)] = (incl - p) + off
                return off + jnp.sum(p)
            lax.fori_loop(0, K // LANES, rbody, jnp.int32(0))
            pltpu.sync_copy(rank_v, rank_hbm.at[s])

        plsc.subcore_barrier()
        pltpu.sync_copy(rank_hbm.at[s], rank_v)

        @plsc.parallel_loop(0, GPQ, 1, unroll=4)
        def mbody(g):
            r = g // WG
            jo = pl.multiple_of((g % WG) * LANES, LANES)
            v = seg_v[r, pl.ds(jo, LANES)]
            inv = plsc.load_gather(rank_v, [zconst, v])
            seg_v[r, pl.ds(jo, LANES)] = inv
            plsc.addupdate_scatter(cnt_v, [zconst, inv], onesf)

        pltpu.sync_copy(seg_v,
                        segs_hbm.at[b, s, pl.ds(r0, QR), pl.ds(0, W)])
        pltpu.sync_copy(cnt_v, cntp_hbm.at[ci, s_local, b, q])

    return rank_kernel


def _build_scatter_kernel(B, C, S, H, W):
    CPT = C // NS
    NSLAB = H // SLAB
    GPS = SLAB * W // LANES
    mesh = plsc.VectorSubcoreMesh(core_axis_name="c", subcore_axis_name="s")

    @functools.partial(
        pl.kernel,
        out_type=jax.ShapeDtypeStruct((B, S, NS, 8, 1, K), jnp.float32),
        mesh=mesh,
        scratch_types=[
            pltpu.VMEM((S * 8 * K,), jnp.float32),
            pltpu.VMEM((2, S, SLAB, W), jnp.int32),
            pltpu.VMEM((2, CPT, SLAB, W), jnp.float32),
            pltpu.SemaphoreType.DMA,
            pltpu.SemaphoreType.DMA,
        ],
        compiler_params=pltpu.CompilerParams(needs_layout_passes=False),
    )
    def scatter_kernel(x_hbm, seg_hbm, sums_hbm,
                       acc_v, seg_v, x_v,
                       sem_seg, sem_x):
        b = lax.axis_index("c")
        cb = lax.axis_index("s")
        zf = jnp.zeros((LANES,), jnp.float32)
        WG = W // LANES

        def zbody(j, carry):
            jo = pl.multiple_of(j * LANES, LANES)
            for blk in range(S * 8):
                acc_v[pl.ds(blk * K + jo, LANES)] = zf
            return carry
        lax.fori_loop(0, K // LANES, zbody, 0)

        def seg_copy(slab, buf):
            r0 = pl.multiple_of(slab * SLAB, SLAB)
            return pltpu.make_async_copy(
                seg_hbm.at[b, pl.ds(0, S), pl.ds(r0, SLAB), pl.ds(0, W)],
                seg_v.at[buf], sem_seg)

        def x_copy(slab, buf):
            r0 = pl.multiple_of(slab * SLAB, SLAB)
            return pltpu.make_async_copy(
                x_hbm.at[b, pl.ds(cb * CPT, CPT), pl.ds(r0, SLAB),
                         pl.ds(0, W)],
                x_v.at[buf], sem_x)

        seg_copy(0, 0).start()
        x_copy(0, 0).start()

        def slab_body(slab, carry):
            parity = lax.rem(slab, 2)

            @pl.when(slab + 1 < NSLAB)
            def _():
                seg_copy(slab + 1, 1 - parity).start()
                x_copy(slab + 1, 1 - parity).start()

            seg_copy(slab, parity).wait()
            x_copy(slab, parity).wait()
            seg_b = seg_v.at[parity]
            x_b = x_v.at[parity]

            @plsc.parallel_loop(0, GPS, 1, unroll=4)
            def gbody(g):
                r = g // WG
                jo = pl.multiple_of((g % WG) * LANES, LANES)
                idxs = [seg_b[s, r, pl.ds(jo, LANES)] for s in range(S)]
                for c in range(CPT):
                    xv = x_b[c, r, pl.ds(jo, LANES)]
                    for s in range(S):
                        plsc.addupdate_scatter(
                            acc_v.at[pl.ds((s * 8 + c) * K, K)],
                            [idxs[s]], xv)
            return carry
        lax.fori_loop(0, NSLAB, slab_body, 0)

        for s in range(S):
            for c in range(8):
                pltpu.sync_copy(acc_v.at[pl.ds((s * 8 + c) * K, K)],
                                sums_hbm.at[b, s, cb, c, 0])

    return scatter_kernel


def _tokens_tc(sums, cnts, cls_token, cls_pos_embed, wt, wt_pad, bias2d):
    B, S, C2, _ = sums.shape
    C = wt.shape[0]
    E = wt.shape[1]

    def body(sums_ref, cnts_ref, cls_ref, clsp_ref, wt_ref, wtp_ref, b_ref,
             out_ref):
        sc = jnp.reshape(sums_ref[...], (C2, K))
        cnt4 = jnp.reshape(cnts_ref[...], (4, K))
        cnt = jnp.sum(cnt4, axis=0, keepdims=True)
        recip = 1.0 / jnp.maximum(cnt, 1.0)
        mean = sc * recip
        body_out = lax.dot_general(mean, wtp_ref[...], (((0,), (0,)), ((), ())),
                                   preferred_element_type=jnp.float32)
        cls_row = jnp.dot(cls_ref[...] + clsp_ref[...], wt_ref[...],
                          preferred_element_type=jnp.float32)
        bias = b_ref[...]
        full = jnp.concatenate([cls_row + bias, body_out + bias], axis=0)
        out_ref[...] = jnp.reshape(full, (1, 1, K + 1, E))

    return pl.pallas_call(
        body,
        grid=(S, B),
        in_specs=[
            pl.BlockSpec((1, 1, C2, K), lambda s, b: (b, s, 0, 0)),
            pl.BlockSpec((1, 1, 1, 4, 1, K),
                         lambda s, b: (s // 2, s % 2, b, 0, 0, 0)),
            pl.BlockSpec((1, C), lambda s, b: (0, 0)),
            pl.BlockSpec((1, C), lambda s, b: (0, 0)),
            pl.BlockSpec((C, E), lambda s, b: (0, 0)),
            pl.BlockSpec((C2, E), lambda s, b: (0, 0)),
            pl.BlockSpec((1, E), lambda s, b: (0, 0)),
        ],
        out_specs=pl.BlockSpec((1, 1, K + 1, E), lambda s, b: (s, b, 0, 0)),
        out_shape=jax.ShapeDtypeStruct((S, B, K + 1, E), jnp.float32),
    )(sums, cnts, cls_token, cls_pos_embed, wt, wt_pad, bias2d)


def kernel(x, segments, cls_token, cls_pos_embed, conv_w, conv_b):
    B, C, H, W = x.shape
    S = segments.shape[1]
    CPT = C // NS

    rank, _, cnts, segs = _build_rank_kernel(B, S, H, W)(segments)
    sums6 = _build_scatter_kernel(B, C, S, H, W)(x, segs)
    sums = sums6.reshape(B, S, NS * 8, K)

    wt = conv_w.T
    E = conv_w.shape[0]
    rows = jnp.arange(C, dtype=jnp.int32)
    pad_rows = (rows // CPT) * 8 + (rows % CPT)
    wt_pad = jnp.zeros((NS * 8, E), jnp.float32).at[pad_rows].set(wt)

    toks = _tokens_tc(sums, cnts, cls_token, cls_pos_embed,
                      wt, wt_pad, conv_b.reshape(1, -1))
    tokens = tuple(toks[i] for i in range(S))
    return (tokens, segs)
